# Initial kernel scaffold; baseline (speedup 1.0000x reference)
import jax
import jax.numpy as jnp
from jax.experimental import pallas as pl


def kernel(heat, edge_weight, W, b, gamma, beta, a_conv, a_act, graph, diff_graph):
    raise NotImplementedError("write your pallas kernel here")


# speedup vs baseline: 3.1862x; 3.1862x over previous
import functools

import jax
import jax.numpy as jnp
from jax import lax
from jax.experimental import pallas as pl
from jax.experimental.pallas import tpu as pltpu
from jax.experimental.pallas import tpu_sc as plsc

_N = 10000
_E = 320000
_D = 128
_NL = 2

_NC = 2
_NS = 16
_NW = _NC * _NS
_BK = 128
_NPAD = _N + 112
_RPT = _NPAD // _NS
_BLOCKS = -(-_E // (_NW * _BK))
_EPAD = _NW * _BLOCKS * _BK
_DW = 16


_CHUNKS = []
_off = 0
while _off < _RPT:
    _CHUNKS.append((_off, min(_BK, _RPT - _off)))
    _off += _BK


def _deg_body(srcp, dstp, zeros_hbm, pat_out_hbm, pat_in_hbm, out_deg,
              sidx_v, didx_v, pout_v, pin_v, acc_sh, sem):
    c = lax.axis_index("c")
    s = lax.axis_index("s")
    wid = c * _NS + s
    base = wid * (_BLOCKS * _BK)
    r0 = s * _RPT
    pltpu.sync_copy(zeros_hbm, pout_v)
    for off, sz in _CHUNKS:
        pltpu.sync_copy(pout_v.at[pl.ds(0, sz)],
                        acc_sh.at[pl.ds(r0 + off, sz)])
    pltpu.sync_copy(pat_out_hbm, pout_v)
    pltpu.sync_copy(pat_in_hbm, pin_v)
    plsc.subcore_barrier()

    def body(j, carry):
        off = base + j * _BK
        pltpu.sync_copy(srcp.at[pl.ds(off, _BK)], sidx_v)
        pltpu.sync_copy(pout_v, acc_sh.at[sidx_v], add=True)
        pltpu.sync_copy(dstp.at[pl.ds(off, _BK)], didx_v)
        pltpu.sync_copy(pin_v, acc_sh.at[didx_v], add=True)
        return carry

    lax.fori_loop(0, _BLOCKS, body, 0)
    plsc.subcore_barrier()
    for off, sz in _CHUNKS:
        pltpu.sync_copy(acc_sh.at[pl.ds(r0 + off, sz)],
                        pout_v.at[pl.ds(0, sz)])
        pltpu.sync_copy(pout_v.at[pl.ds(0, sz)],
                        out_deg.at[c].at[pl.ds(r0 + off, sz)])


def _agg_body(table, srcp, dstp, zeros_hbm, out_agg,
              sidx_v, didx_v, rows_v, acc_sh, sem):
    c = lax.axis_index("c")
    s = lax.axis_index("s")
    wid = c * _NS + s
    base = wid * (_BLOCKS * _BK)
    r0 = s * _RPT
    pltpu.sync_copy(zeros_hbm, rows_v)
    for off, sz in _CHUNKS:
        pltpu.sync_copy(rows_v.at[pl.ds(0, sz)],
                        acc_sh.at[pl.ds(r0 + off, sz)])
    plsc.subcore_barrier()

    def body(j, carry):
        off = base + j * _BK
        pltpu.sync_copy(srcp.at[pl.ds(off, _BK)], sidx_v)
        pltpu.async_copy(table.at[sidx_v], rows_v, sem).wait()
        pltpu.sync_copy(dstp.at[pl.ds(off, _BK)], didx_v)
        pltpu.sync_copy(rows_v, acc_sh.at[didx_v], add=True)
        return carry

    lax.fori_loop(0, _BLOCKS, body, 0)
    plsc.subcore_barrier()
    for off, sz in _CHUNKS:
        pltpu.sync_copy(acc_sh.at[pl.ds(r0 + off, sz)],
                        rows_v.at[pl.ds(0, sz)])
        pltpu.sync_copy(rows_v.at[pl.ds(0, sz)],
                        out_agg.at[c].at[pl.ds(r0 + off, sz)])


@functools.lru_cache(maxsize=None)
def _sc_kernels():
    mesh = plsc.VectorSubcoreMesh(
        core_axis_name="c", subcore_axis_name="s",
        num_cores=_NC, num_subcores=_NS)
    deg_kernel = pl.kernel(
        _deg_body,
        out_type=jax.ShapeDtypeStruct((_NC, _NPAD, _D), jnp.float32),
        mesh=mesh,
        scratch_types=[
            pltpu.VMEM((_BK,), jnp.int32),
            pltpu.VMEM((_BK,), jnp.int32),
            pltpu.VMEM((_BK, _D), jnp.float32),
            pltpu.VMEM((_BK, _D), jnp.float32),
            pltpu.VMEM_SHARED((_NPAD, _D), jnp.float32),
            pltpu.SemaphoreType.DMA,
        ],
    )
    agg_kernel = pl.kernel(
        _agg_body,
        out_type=jax.ShapeDtypeStruct((_NC, _NPAD, _D), jnp.float32),
        mesh=mesh,
        compiler_params=pltpu.CompilerParams(use_tc_tiling_on_sc=False),
        scratch_types=[
            pltpu.VMEM((_BK,), jnp.int32),
            pltpu.VMEM((_BK,), jnp.int32),
            pltpu.VMEM((_BK, _D), jnp.float32),
            pltpu.VMEM_SHARED((_NPAD, _D), jnp.float32),
            pltpu.SemaphoreType.DMA,
        ],
    )
    return deg_kernel, agg_kernel


def _prep_body(heat_ref, deg_ref, hs_ref, nd_ref, ns_ref):
    deg_out = deg_ref[0, :_N, 0:1] + deg_ref[1, :_N, 0:1]
    deg_in = deg_ref[0, :_N, 64:65] + deg_ref[1, :_N, 64:65]
    ns_col = lax.rsqrt(jnp.maximum(deg_out, 1.0))
    nd_col = lax.rsqrt(jnp.maximum(deg_in, 1.0))
    ns_ref[...] = ns_col
    nd_ref[...] = nd_col
    hs_ref[:_N, :] = heat_ref[...] * ns_col
    hs_ref[_N:, :] = jnp.zeros((_NPAD - _N, _D), jnp.float32)


def _dense_body(aggp_ref, nd_ref, ns_ref, w_ref, b_ref,
                gam_ref, bet_ref, ac_ref, aa_ref, hs_ref, h_ref):
    agg = aggp_ref[0, :_N, :] + aggp_ref[1, :_N, :]
    x = agg * nd_ref[...]
    h = jnp.dot(x, w_ref[...], preferred_element_type=jnp.float32) + b_ref[...]
    ac = ac_ref[0, 0]
    h = jnp.where(h >= 0.0, h, ac * h)
    mu = jnp.mean(h, axis=0, keepdims=True)
    var = jnp.mean((h - mu) * (h - mu), axis=0, keepdims=True)
    h = (h - mu) * lax.rsqrt(var + 1e-5) * gam_ref[...] + bet_ref[...]
    aa = aa_ref[0, 0]
    h = jnp.where(h >= 0.0, h, aa * h)
    h_ref[...] = h
    hs_ref[:_N, :] = h * ns_ref[...]
    hs_ref[_N:, :] = jnp.zeros((_NPAD - _N, _D), jnp.float32)


_prep_call = pl.pallas_call(
    _prep_body,
    out_shape=(
        jax.ShapeDtypeStruct((_NPAD, _D), jnp.float32),
        jax.ShapeDtypeStruct((_N, 1), jnp.float32),
        jax.ShapeDtypeStruct((_N, 1), jnp.float32),
    ),
)

_dense_call = pl.pallas_call(
    _dense_body,
    out_shape=(
        jax.ShapeDtypeStruct((_NPAD, _D), jnp.float32),
        jax.ShapeDtypeStruct((_N, _D), jnp.float32),
    ),
)


def kernel(heat, edge_weight, W, b, gamma, beta, a_conv, a_act, graph, diff_graph):
    src = graph[0].astype(jnp.int32)
    dst = graph[1].astype(jnp.int32)
    pad = _EPAD - _E
    padv = jnp.full((pad,), _N, jnp.int32)
    srcp = jnp.concatenate([src, padv])
    dstp = jnp.concatenate([dst, padv])

    zeros_rows = jnp.zeros((_BK, _D), jnp.float32)
    half = jnp.concatenate([jnp.ones((_BK, _D // 2), jnp.float32),
                            jnp.zeros((_BK, _D // 2), jnp.float32)], axis=1)
    pat_out = half
    pat_in = 1.0 - half

    deg_kernel, agg_kernel = _sc_kernels()
    degp = deg_kernel(srcp, dstp, zeros_rows, pat_out, pat_in)
    hs0, nd_col, ns_col = _prep_call(heat, degp)

    def layer(hs, xs):
        w, bv, gv, betav, acv, aav = xs
        aggp = agg_kernel(hs, srcp, dstp, zeros_rows)
        hs_next, h = _dense_call(aggp, nd_col, ns_col, w, bv, gv, betav,
                                 acv, aav)
        return hs_next, h

    _, ys = lax.scan(
        layer, hs0,
        (W, b.reshape(_NL, 1, _D), gamma.reshape(_NL, 1, _D),
         beta.reshape(_NL, 1, _D), a_conv.reshape(_NL, 1, 1),
         a_act.reshape(_NL, 1, 1)))
    return ys[_NL - 1]


# speedup vs baseline: 3.4545x; 1.0842x over previous
import functools

import jax
import jax.numpy as jnp
from jax import lax
from jax.experimental import pallas as pl
from jax.experimental.pallas import tpu as pltpu
from jax.experimental.pallas import tpu_sc as plsc

_N = 10000
_E = 320000
_D = 128
_NL = 2

_NC = 2
_NS = 16
_NW = _NC * _NS
_BK = 128
_NPAD = _N + 112
_RPT = _NPAD // _NS
_BLOCKS = 80
_EPAD = _NW * _BLOCKS * _BK
_PAIRS = _BLOCKS // 2
_DW = 16


_CHUNKS = []
_off = 0
while _off < _RPT:
    _CHUNKS.append((_off, min(_BK, _RPT - _off)))
    _off += _BK


def _deg_body(srcp, dstp, zeros_hbm, pat_out_hbm, pat_in_hbm, out_deg,
              sidx_v, didx_v, pout_v, pin_v, acc_sh, sem):
    c = lax.axis_index("c")
    s = lax.axis_index("s")
    wid = c * _NS + s
    base = wid * (_BLOCKS * _BK)
    r0 = s * _RPT
    pltpu.sync_copy(zeros_hbm, pout_v)
    for off, sz in _CHUNKS:
        pltpu.sync_copy(pout_v.at[pl.ds(0, sz)],
                        acc_sh.at[pl.ds(r0 + off, sz)])
    pltpu.sync_copy(pat_out_hbm, pout_v)
    pltpu.sync_copy(pat_in_hbm, pin_v)
    plsc.subcore_barrier()

    def body(j, carry):
        off = base + j * _BK
        pltpu.sync_copy(srcp.at[pl.ds(off, _BK)], sidx_v)
        pltpu.sync_copy(pout_v, acc_sh.at[sidx_v], add=True)
        pltpu.sync_copy(dstp.at[pl.ds(off, _BK)], didx_v)
        pltpu.sync_copy(pin_v, acc_sh.at[didx_v], add=True)
        return carry

    lax.fori_loop(0, _BLOCKS, body, 0)
    plsc.subcore_barrier()
    for off, sz in _CHUNKS:
        pltpu.sync_copy(acc_sh.at[pl.ds(r0 + off, sz)],
                        pout_v.at[pl.ds(0, sz)])
        pltpu.sync_copy(pout_v.at[pl.ds(0, sz)],
                        out_deg.at[c].at[pl.ds(r0 + off, sz)])


def _agg_body(table, src2d, dstp, zeros_hbm, out_agg,
              sidx_all, didx0, didx1, rows0, rows1,
              acc_sh, gsem0, gsem1, dsem0, dsem1):
    c = lax.axis_index("c")
    s = lax.axis_index("s")
    wid = c * _NS + s
    rb = wid * _BLOCKS
    eb = rb * _BK
    r0 = s * _RPT
    pltpu.sync_copy(zeros_hbm, rows0)
    for off, sz in _CHUNKS:
        pltpu.sync_copy(rows0.at[pl.ds(0, sz)],
                        acc_sh.at[pl.ds(r0 + off, sz)])
    pltpu.sync_copy(src2d.at[pl.ds(rb, _BLOCKS)], sidx_all)
    plsc.subcore_barrier()

    pltpu.async_copy(dstp.at[pl.ds(eb, _BK)], didx0, dsem0)
    pltpu.async_copy(table.at[sidx_all.at[0]], rows0, gsem0)
    pltpu.async_copy(dstp.at[pl.ds(eb + _BK, _BK)], didx1, dsem1)
    pltpu.async_copy(table.at[sidx_all.at[1]], rows1, gsem1)

    def lane(j, rows, didx, gsem, dsem, prefetch):
        pltpu.make_async_copy(dstp.at[pl.ds(eb + j * _BK, _BK)],
                              didx, dsem).wait()
        pltpu.make_async_copy(table.at[sidx_all.at[j]], rows, gsem).wait()
        pltpu.sync_copy(rows, acc_sh.at[didx], add=True)

        @pl.when(prefetch)
        def _():
            pltpu.async_copy(dstp.at[pl.ds(eb + (j + 2) * _BK, _BK)],
                             didx, dsem)
            pltpu.async_copy(table.at[sidx_all.at[j + 2]], rows, gsem)

    def body(k, carry):
        lane(2 * k, rows0, didx0, gsem0, dsem0, k < _PAIRS - 1)
        lane(2 * k + 1, rows1, didx1, gsem1, dsem1, k < _PAIRS - 1)
        return carry

    lax.fori_loop(0, _PAIRS, body, 0)
    plsc.subcore_barrier()
    for off, sz in _CHUNKS:
        pltpu.sync_copy(acc_sh.at[pl.ds(r0 + off, sz)],
                        rows0.at[pl.ds(0, sz)])
        pltpu.sync_copy(rows0.at[pl.ds(0, sz)],
                        out_agg.at[c].at[pl.ds(r0 + off, sz)])


@functools.lru_cache(maxsize=None)
def _sc_kernels():
    mesh = plsc.VectorSubcoreMesh(
        core_axis_name="c", subcore_axis_name="s",
        num_cores=_NC, num_subcores=_NS)
    deg_kernel = pl.kernel(
        _deg_body,
        out_type=jax.ShapeDtypeStruct((_NC, _NPAD, _D), jnp.float32),
        mesh=mesh,
        scratch_types=[
            pltpu.VMEM((_BK,), jnp.int32),
            pltpu.VMEM((_BK,), jnp.int32),
            pltpu.VMEM((_BK, _D), jnp.float32),
            pltpu.VMEM((_BK, _D), jnp.float32),
            pltpu.VMEM_SHARED((_NPAD, _D), jnp.float32),
            pltpu.SemaphoreType.DMA,
        ],
    )
    agg_kernel = pl.kernel(
        _agg_body,
        out_type=jax.ShapeDtypeStruct((_NC, _NPAD, _D), jnp.float32),
        mesh=mesh,
        scratch_types=[
            pltpu.VMEM((_BLOCKS, _BK), jnp.int32),
            pltpu.VMEM((_BK,), jnp.int32),
            pltpu.VMEM((_BK,), jnp.int32),
            pltpu.VMEM((_BK, _D), jnp.float32),
            pltpu.VMEM((_BK, _D), jnp.float32),
            pltpu.VMEM_SHARED((_NPAD, _D), jnp.float32),
            pltpu.SemaphoreType.DMA,
            pltpu.SemaphoreType.DMA,
            pltpu.SemaphoreType.DMA,
            pltpu.SemaphoreType.DMA,
        ],
    )
    return deg_kernel, agg_kernel


def _prep_body(heat_ref, deg_ref, hs_ref, nd_ref, ns_ref):
    deg_out = deg_ref[0, :_N, 0:1] + deg_ref[1, :_N, 0:1]
    deg_in = deg_ref[0, :_N, 64:65] + deg_ref[1, :_N, 64:65]
    ns_col = lax.rsqrt(jnp.maximum(deg_out, 1.0))
    nd_col = lax.rsqrt(jnp.maximum(deg_in, 1.0))
    ns_ref[...] = ns_col
    nd_ref[...] = nd_col
    hs_ref[:_N, :] = heat_ref[...] * ns_col
    hs_ref[_N:, :] = jnp.zeros((_NPAD - _N, _D), jnp.float32)


def _dense_body(aggp_ref, nd_ref, ns_ref, w_ref, b_ref,
                gam_ref, bet_ref, ac_ref, aa_ref, hs_ref, h_ref):
    agg = aggp_ref[0, :_N, :] + aggp_ref[1, :_N, :]
    x = agg * nd_ref[...]
    h = jnp.dot(x, w_ref[...], preferred_element_type=jnp.float32) + b_ref[...]
    ac = ac_ref[0, 0]
    h = jnp.where(h >= 0.0, h, ac * h)
    mu = jnp.mean(h, axis=0, keepdims=True)
    var = jnp.mean((h - mu) * (h - mu), axis=0, keepdims=True)
    h = (h - mu) * lax.rsqrt(var + 1e-5) * gam_ref[...] + bet_ref[...]
    aa = aa_ref[0, 0]
    h = jnp.where(h >= 0.0, h, aa * h)
    h_ref[...] = h
    hs_ref[:_N, :] = h * ns_ref[...]
    hs_ref[_N:, :] = jnp.zeros((_NPAD - _N, _D), jnp.float32)


_prep_call = pl.pallas_call(
    _prep_body,
    out_shape=(
        jax.ShapeDtypeStruct((_NPAD, _D), jnp.float32),
        jax.ShapeDtypeStruct((_N, 1), jnp.float32),
        jax.ShapeDtypeStruct((_N, 1), jnp.float32),
    ),
)

_dense_call = pl.pallas_call(
    _dense_body,
    out_shape=(
        jax.ShapeDtypeStruct((_NPAD, _D), jnp.float32),
        jax.ShapeDtypeStruct((_N, _D), jnp.float32),
    ),
)


def kernel(heat, edge_weight, W, b, gamma, beta, a_conv, a_act, graph, diff_graph):
    src = graph[0].astype(jnp.int32)
    dst = graph[1].astype(jnp.int32)
    pad = _EPAD - _E
    padv = jnp.full((pad,), _N, jnp.int32)
    srcp = jnp.concatenate([src, padv])
    dstp = jnp.concatenate([dst, padv])

    zeros_rows = jnp.zeros((_BK, _D), jnp.float32)
    half = jnp.concatenate([jnp.ones((_BK, _D // 2), jnp.float32),
                            jnp.zeros((_BK, _D // 2), jnp.float32)], axis=1)
    pat_out = half
    pat_in = 1.0 - half

    deg_kernel, agg_kernel = _sc_kernels()
    degp = deg_kernel(srcp, dstp, zeros_rows, pat_out, pat_in)
    hs0, nd_col, ns_col = _prep_call(heat, degp)

    src2d = srcp.reshape(_EPAD // _BK, _BK)

    def layer(hs, xs):
        w, bv, gv, betav, acv, aav = xs
        aggp = agg_kernel(hs, src2d, dstp, zeros_rows)
        hs_next, h = _dense_call(aggp, nd_col, ns_col, w, bv, gv, betav,
                                 acv, aav)
        return hs_next, h

    _, ys = lax.scan(
        layer, hs0,
        (W, b.reshape(_NL, 1, _D), gamma.reshape(_NL, 1, _D),
         beta.reshape(_NL, 1, _D), a_conv.reshape(_NL, 1, 1),
         a_act.reshape(_NL, 1, 1)))
    return ys[_NL - 1]
